# baseline (speedup 1.0000x reference)
import functools

import jax
import jax.numpy as jnp
from jax import lax
from jax.experimental import pallas as pl
from jax.experimental.pallas import tpu as pltpu
from jax.experimental.pallas import tpu_sc as plsc

T = 2048
D = 1024
E = 64
F = 512
BT = 32
NB = 128
NBT = NB * BT


NBV = NB + 8


def _router_body(x_ref, wg_ref, logits_ref, pos_ref, be_ref):
    x = x_ref[...]
    wg = wg_ref[...]
    logits = jnp.dot(x, wg, preferred_element_type=jnp.float32)
    logits_ref[...] = logits
    m = jnp.max(logits, axis=1, keepdims=True)
    col = lax.broadcasted_iota(jnp.int32, (T, E), 1)
    eid = jnp.min(jnp.where(logits == m, col, E), axis=1, keepdims=True)
    oh = (col == eid).astype(jnp.float32)
    r = lax.broadcasted_iota(jnp.int32, (T, T), 0)
    c2 = lax.broadcasted_iota(jnp.int32, (T, T), 1)
    ltri = (c2 < r).astype(jnp.float32)
    rank_full = jnp.dot(ltri, oh, preferred_element_type=jnp.float32)
    rank = jnp.sum(rank_full * oh, axis=1, keepdims=True)
    counts = jnp.sum(oh, axis=0, keepdims=True)
    nblk = jnp.floor((counts + (BT - 1)) * (1.0 / BT))
    re = lax.broadcasted_iota(jnp.int32, (E, E), 0)
    ce = lax.broadcasted_iota(jnp.int32, (E, E), 1)
    tri = (re <= ce).astype(jnp.float32)
    blk_cum = jnp.dot(nblk, tri, preferred_element_type=jnp.float32)
    pstart = (blk_cum - nblk) * BT
    pos = jnp.sum(oh * pstart, axis=1, keepdims=True) + rank
    pos_ref[...] = pos.astype(jnp.int32)
    rowi = lax.broadcasted_iota(jnp.int32, (NBV, E), 0).astype(jnp.float32)
    becnt = jnp.sum((jnp.broadcast_to(blk_cum, (NBV, E)) <= rowi)
                    .astype(jnp.float32), axis=1, keepdims=True)
    total = blk_cum[:, E - 1:E]
    rowi1 = lax.broadcasted_iota(jnp.int32, (NBV, 1), 0).astype(jnp.float32)
    bevec = jnp.where(rowi1 == NB, jnp.broadcast_to(total, (NBV, 1)),
                      jnp.minimum(becnt, E - 1))
    be_ref[...] = bevec.astype(jnp.int32)


def _router(x, wg):
    return pl.pallas_call(
        _router_body,
        out_shape=[
            jax.ShapeDtypeStruct((T, E), jnp.float32),
            jax.ShapeDtypeStruct((T, 1), jnp.int32),
            jax.ShapeDtypeStruct((NBV, 1), jnp.int32),
        ],
    )(x, wg)


def _sc_gather_body(nch, ch, bpw, table_ref, idx_ref, out_ref, idx_v, rows_v, sem):
    info = plsc.get_sparse_core_info()
    nc = info.num_cores
    wid = lax.axis_index("s") * nc + lax.axis_index("c")
    for c in range(nch):
        base = wid * bpw + c * ch
        pltpu.sync_copy(idx_ref.at[pl.ds(base, ch)], idx_v)
        pltpu.async_copy(table_ref.at[idx_v], rows_v, sem).wait()
        pltpu.sync_copy(rows_v, out_ref.at[pl.ds(base, ch)])


def _sc_scatter_body(ch, bpw, rows_hbm, pos_hbm, out_ref, idx_v, rows_v, sem):
    info = plsc.get_sparse_core_info()
    nc = info.num_cores
    wid = lax.axis_index("s") * nc + lax.axis_index("c")
    base = wid * bpw
    pltpu.sync_copy(pos_hbm.at[pl.ds(base, ch)], idx_v)
    pltpu.sync_copy(rows_hbm.at[pl.ds(base, ch)], rows_v)
    pltpu.async_copy(rows_v, out_ref.at[idx_v], sem).wait()


def _sc_scatter(rows, pos, nbt):
    b, d = rows.shape
    info = plsc.get_sparse_core_info()
    nw = info.num_cores * info.num_subcores
    bpw = b // nw
    mesh = plsc.VectorSubcoreMesh(core_axis_name="c", subcore_axis_name="s")
    fn = pl.kernel(
        functools.partial(_sc_scatter_body, bpw, bpw),
        mesh=mesh,
        out_type=jax.ShapeDtypeStruct((nbt, d), rows.dtype),
        scratch_types=[
            pltpu.VMEM((bpw,), jnp.int32),
            pltpu.VMEM((bpw, d), rows.dtype),
            pltpu.SemaphoreType.DMA,
        ],
    )
    return fn(rows, pos)


def _sc_gather(table, idx):
    b = idx.shape[0]
    info = plsc.get_sparse_core_info()
    nw = info.num_cores * info.num_subcores
    bpw = b // nw
    ch = min(bpw, 64)
    nch = bpw // ch
    mesh = plsc.VectorSubcoreMesh(core_axis_name="c", subcore_axis_name="s")
    fn = pl.kernel(
        functools.partial(_sc_gather_body, nch, ch, bpw),
        mesh=mesh,
        out_type=jax.ShapeDtypeStruct((b, table.shape[1]), table.dtype),
        scratch_types=[
            pltpu.VMEM((ch,), jnp.int32),
            pltpu.VMEM((ch, table.shape[1]), table.dtype),
            pltpu.SemaphoreType.DMA,
        ],
    )
    return fn(table, idx)


def _ffn_body(be_ref, tb_ref, x_ref, w1_ref, w3_ref, w2_ref, y_ref):
    i = pl.program_id(0)

    @pl.when(i < tb_ref[0])
    def _():
        x = x_ref[...]
        h1 = jnp.dot(x, w1_ref[0], preferred_element_type=jnp.float32)
        h3 = jnp.dot(x, w3_ref[0], preferred_element_type=jnp.float32)
        h = h1 * (1.0 / (1.0 + jnp.exp(-h1))) * h3
        y_ref[...] = jnp.dot(h, w2_ref[0], preferred_element_type=jnp.float32)


def _ffn(x_pad, be, tb, W1, W3, W2):
    grid_spec = pltpu.PrefetchScalarGridSpec(
        num_scalar_prefetch=2,
        grid=(NB,),
        in_specs=[
            pl.BlockSpec((BT, D), lambda i, be, tb: (i, 0)),
            pl.BlockSpec((1, D, F), lambda i, be, tb: (be[i], 0, 0)),
            pl.BlockSpec((1, D, F), lambda i, be, tb: (be[i], 0, 0)),
            pl.BlockSpec((1, F, D), lambda i, be, tb: (be[i], 0, 0)),
        ],
        out_specs=pl.BlockSpec((BT, D), lambda i, be, tb: (i, 0)),
    )
    return pl.pallas_call(
        _ffn_body,
        grid_spec=grid_spec,
        out_shape=jax.ShapeDtypeStruct((NBT, D), jnp.float32),
    )(be, tb, x_pad, W1, W3, W2)


def kernel(hidden_states, Wg, W1, W3, W2):
    b, s, d = hidden_states.shape
    x = hidden_states.reshape(-1, d)

    logits, pos2, bevec = _router(x, Wg)
    pos = pos2[:, 0]

    be = bevec[:NB, 0]
    tb = bevec[NB, 0][None]

    x_pad = _sc_scatter(x, pos, NBT)
    y_pad = _ffn(x_pad, be, tb, W1, W3, W2)
    out = _sc_gather(y_pad, pos.astype(jnp.int32))

    return out.reshape(b, s, d), logits


# speedup vs baseline: 1.2634x; 1.2634x over previous
import functools

import jax
import jax.numpy as jnp
from jax import lax
from jax.experimental import pallas as pl
from jax.experimental.pallas import tpu as pltpu
from jax.experimental.pallas import tpu_sc as plsc

T = 2048
D = 1024
E = 64
F = 512
BT = 128
NB = 80
NBT = NB * BT


NBV = NB + 8


def _router_body(x_ref, wg_ref, logits_ref, pos_ref, be_ref):
    x = x_ref[...]
    wg = wg_ref[...]
    logits = jnp.dot(x, wg, preferred_element_type=jnp.float32)
    logits_ref[...] = logits
    m = jnp.max(logits, axis=1, keepdims=True)
    col = lax.broadcasted_iota(jnp.int32, (T, E), 1)
    eid = jnp.min(jnp.where(logits == m, col, E), axis=1, keepdims=True)
    oh = (col == eid).astype(jnp.float32)
    r = lax.broadcasted_iota(jnp.int32, (T, T), 0)
    c2 = lax.broadcasted_iota(jnp.int32, (T, T), 1)
    ltri = (c2 < r).astype(jnp.float32)
    rank_full = jnp.dot(ltri, oh, preferred_element_type=jnp.float32)
    rank = jnp.sum(rank_full * oh, axis=1, keepdims=True)
    counts = jnp.sum(oh, axis=0, keepdims=True)
    nblk = jnp.floor((counts + (BT - 1)) * (1.0 / BT))
    re = lax.broadcasted_iota(jnp.int32, (E, E), 0)
    ce = lax.broadcasted_iota(jnp.int32, (E, E), 1)
    tri = (re <= ce).astype(jnp.float32)
    blk_cum = jnp.dot(nblk, tri, preferred_element_type=jnp.float32)
    pstart = (blk_cum - nblk) * BT
    pos = jnp.sum(oh * pstart, axis=1, keepdims=True) + rank
    pos_ref[...] = pos.astype(jnp.int32)
    rowi = lax.broadcasted_iota(jnp.int32, (NBV, E), 0).astype(jnp.float32)
    becnt = jnp.sum((jnp.broadcast_to(blk_cum, (NBV, E)) <= rowi)
                    .astype(jnp.float32), axis=1, keepdims=True)
    total = blk_cum[:, E - 1:E]
    rowi1 = lax.broadcasted_iota(jnp.int32, (NBV, 1), 0).astype(jnp.float32)
    bevec = jnp.where(rowi1 == NB, jnp.broadcast_to(total, (NBV, 1)),
                      jnp.minimum(becnt, E - 1))
    be_ref[...] = bevec.astype(jnp.int32)


def _router(x, wg):
    return pl.pallas_call(
        _router_body,
        out_shape=[
            jax.ShapeDtypeStruct((T, E), jnp.float32),
            jax.ShapeDtypeStruct((T, 1), jnp.int32),
            jax.ShapeDtypeStruct((NBV, 1), jnp.int32),
        ],
    )(x, wg)


def _sc_gather_body(nch, ch, bpw, table_ref, idx_ref, out_ref, idx_v, rows_v, sem):
    info = plsc.get_sparse_core_info()
    nc = info.num_cores
    wid = lax.axis_index("s") * nc + lax.axis_index("c")
    for c in range(nch):
        base = wid * bpw + c * ch
        pltpu.sync_copy(idx_ref.at[pl.ds(base, ch)], idx_v)
        pltpu.async_copy(table_ref.at[idx_v], rows_v, sem).wait()
        pltpu.sync_copy(rows_v, out_ref.at[pl.ds(base, ch)])


def _sc_scatter_body(ch, bpw, rows_hbm, pos_hbm, out_ref, idx_v, rows_v, sem):
    info = plsc.get_sparse_core_info()
    nc = info.num_cores
    wid = lax.axis_index("s") * nc + lax.axis_index("c")
    base = wid * bpw
    pltpu.sync_copy(pos_hbm.at[pl.ds(base, ch)], idx_v)
    pltpu.sync_copy(rows_hbm.at[pl.ds(base, ch)], rows_v)
    pltpu.async_copy(rows_v, out_ref.at[idx_v], sem).wait()


def _sc_scatter(rows, pos, nbt):
    b, d = rows.shape
    info = plsc.get_sparse_core_info()
    nw = info.num_cores * info.num_subcores
    bpw = b // nw
    mesh = plsc.VectorSubcoreMesh(core_axis_name="c", subcore_axis_name="s")
    fn = pl.kernel(
        functools.partial(_sc_scatter_body, bpw, bpw),
        mesh=mesh,
        out_type=jax.ShapeDtypeStruct((nbt, d), rows.dtype),
        scratch_types=[
            pltpu.VMEM((bpw,), jnp.int32),
            pltpu.VMEM((bpw, d), rows.dtype),
            pltpu.SemaphoreType.DMA,
        ],
    )
    return fn(rows, pos)


def _sc_gather(table, idx):
    b = idx.shape[0]
    info = plsc.get_sparse_core_info()
    nw = info.num_cores * info.num_subcores
    bpw = b // nw
    ch = min(bpw, 64)
    nch = bpw // ch
    mesh = plsc.VectorSubcoreMesh(core_axis_name="c", subcore_axis_name="s")
    fn = pl.kernel(
        functools.partial(_sc_gather_body, nch, ch, bpw),
        mesh=mesh,
        out_type=jax.ShapeDtypeStruct((b, table.shape[1]), table.dtype),
        scratch_types=[
            pltpu.VMEM((ch,), jnp.int32),
            pltpu.VMEM((ch, table.shape[1]), table.dtype),
            pltpu.SemaphoreType.DMA,
        ],
    )
    return fn(table, idx)


def _ffn_body(be_ref, tb_ref, x_ref, w1_ref, w3_ref, w2_ref, y_ref):
    i = pl.program_id(0)

    @pl.when(i < tb_ref[0])
    def _():
        x = x_ref[...]
        h1 = jnp.dot(x, w1_ref[0], preferred_element_type=jnp.float32)
        h3 = jnp.dot(x, w3_ref[0], preferred_element_type=jnp.float32)
        h = h1 * (1.0 / (1.0 + jnp.exp(-h1))) * h3
        y_ref[...] = jnp.dot(h, w2_ref[0], preferred_element_type=jnp.float32)


def _ffn(x_pad, be, tb, W1, W3, W2):
    grid_spec = pltpu.PrefetchScalarGridSpec(
        num_scalar_prefetch=2,
        grid=(NB,),
        in_specs=[
            pl.BlockSpec((BT, D), lambda i, be, tb: (i, 0)),
            pl.BlockSpec((1, D, F), lambda i, be, tb: (be[i], 0, 0)),
            pl.BlockSpec((1, D, F), lambda i, be, tb: (be[i], 0, 0)),
            pl.BlockSpec((1, F, D), lambda i, be, tb: (be[i], 0, 0)),
        ],
        out_specs=pl.BlockSpec((BT, D), lambda i, be, tb: (i, 0)),
    )
    return pl.pallas_call(
        _ffn_body,
        grid_spec=grid_spec,
        out_shape=jax.ShapeDtypeStruct((NBT, D), jnp.float32),
    )(be, tb, x_pad, W1, W3, W2)


def kernel(hidden_states, Wg, W1, W3, W2):
    b, s, d = hidden_states.shape
    x = hidden_states.reshape(-1, d)

    logits, pos2, bevec = _router(x, Wg)
    pos = pos2[:, 0]

    be = bevec[:NB, 0]
    tb = bevec[NB, 0][None]

    x_pad = _sc_scatter(x, pos, NBT)
    y_pad = _ffn(x_pad, be, tb, W1, W3, W2)
    out = _sc_gather(y_pad, pos.astype(jnp.int32))

    return out.reshape(b, s, d), logits


# speedup vs baseline: 1.3961x; 1.1050x over previous
import functools

import jax
import jax.numpy as jnp
from jax import lax
from jax.experimental import pallas as pl
from jax.experimental.pallas import tpu as pltpu
from jax.experimental.pallas import tpu_sc as plsc

T = 2048
D = 1024
E = 64
F = 512
BT = 64
NB = 96
NBT = NB * BT


NBV = NB + 8


def _router_body(x_ref, wg_ref, logits_ref, pos_ref, be_ref):
    x = x_ref[...]
    wg = wg_ref[...]
    logits = jnp.dot(x, wg, preferred_element_type=jnp.float32)
    logits_ref[...] = logits
    m = jnp.max(logits, axis=1, keepdims=True)
    col = lax.broadcasted_iota(jnp.int32, (T, E), 1)
    eid = jnp.min(jnp.where(logits == m, col, E), axis=1, keepdims=True)
    oh = (col == eid).astype(jnp.float32)
    ch = 256
    r = lax.broadcasted_iota(jnp.int32, (ch, ch), 0)
    c2 = lax.broadcasted_iota(jnp.int32, (ch, ch), 1)
    ltri = (c2 < r).astype(jnp.float32)
    carry = jnp.zeros((1, E), jnp.float32)
    ranks = []
    for k in range(T // ch):
        ohk = oh[k * ch:(k + 1) * ch]
        cum = jnp.dot(ltri, ohk, preferred_element_type=jnp.float32) + carry
        ranks.append(jnp.sum(cum * ohk, axis=1, keepdims=True))
        carry = carry + jnp.sum(ohk, axis=0, keepdims=True)
    rank = jnp.concatenate(ranks, axis=0)
    counts = carry
    nblk = jnp.floor((counts + (BT - 1)) * (1.0 / BT))
    re = lax.broadcasted_iota(jnp.int32, (E, E), 0)
    ce = lax.broadcasted_iota(jnp.int32, (E, E), 1)
    tri = (re <= ce).astype(jnp.float32)
    blk_cum = jnp.dot(nblk, tri, preferred_element_type=jnp.float32)
    pstart = (blk_cum - nblk) * BT
    pos = jnp.sum(oh * pstart, axis=1, keepdims=True) + rank
    pos_ref[...] = pos.astype(jnp.int32)
    rowi = lax.broadcasted_iota(jnp.int32, (NBV, E), 0).astype(jnp.float32)
    becnt = jnp.sum((jnp.broadcast_to(blk_cum, (NBV, E)) <= rowi)
                    .astype(jnp.float32), axis=1, keepdims=True)
    total = blk_cum[:, E - 1:E]
    lastexp = jnp.max(ce[:1].astype(jnp.float32) * (nblk > 0), axis=1,
                      keepdims=True)
    rowi1 = lax.broadcasted_iota(jnp.int32, (NBV, 1), 0).astype(jnp.float32)
    bevec = jnp.where(rowi1 < total, jnp.minimum(becnt, E - 1),
                      jnp.broadcast_to(lastexp, (NBV, 1)))
    bevec = jnp.where(rowi1 == NB, jnp.broadcast_to(total, (NBV, 1)), bevec)
    be_ref[...] = bevec.astype(jnp.int32)


def _router(x, wg):
    return pl.pallas_call(
        _router_body,
        out_shape=[
            jax.ShapeDtypeStruct((T, E), jnp.float32),
            jax.ShapeDtypeStruct((T, 1), jnp.int32),
            jax.ShapeDtypeStruct((NBV, 1), jnp.int32),
        ],
    )(x, wg)


def _sc_gather_body(nch, ch, bpw, table_ref, idx_ref, out_ref, idx_v, rows_v, sem):
    info = plsc.get_sparse_core_info()
    nc = info.num_cores
    wid = lax.axis_index("s") * nc + lax.axis_index("c")
    for c in range(nch):
        base = wid * bpw + c * ch
        pltpu.sync_copy(idx_ref.at[pl.ds(base, ch)], idx_v)
        pltpu.async_copy(table_ref.at[idx_v], rows_v, sem).wait()
        pltpu.sync_copy(rows_v, out_ref.at[pl.ds(base, ch)])


def _sc_scatter_body(ch, bpw, rows_hbm, pos_hbm, out_ref, idx_v, rows_v, sem):
    info = plsc.get_sparse_core_info()
    nc = info.num_cores
    wid = lax.axis_index("s") * nc + lax.axis_index("c")
    base = wid * bpw
    pltpu.sync_copy(pos_hbm.at[pl.ds(base, ch)], idx_v)
    pltpu.sync_copy(rows_hbm.at[pl.ds(base, ch)], rows_v)
    pltpu.async_copy(rows_v, out_ref.at[idx_v], sem).wait()


def _sc_scatter(rows, pos, nbt):
    b, d = rows.shape
    info = plsc.get_sparse_core_info()
    nw = info.num_cores * info.num_subcores
    bpw = b // nw
    mesh = plsc.VectorSubcoreMesh(core_axis_name="c", subcore_axis_name="s")
    fn = pl.kernel(
        functools.partial(_sc_scatter_body, bpw, bpw),
        mesh=mesh,
        out_type=jax.ShapeDtypeStruct((nbt, d), rows.dtype),
        scratch_types=[
            pltpu.VMEM((bpw,), jnp.int32),
            pltpu.VMEM((bpw, d), rows.dtype),
            pltpu.SemaphoreType.DMA,
        ],
    )
    return fn(rows, pos)


def _sc_gather(table, idx):
    b = idx.shape[0]
    info = plsc.get_sparse_core_info()
    nw = info.num_cores * info.num_subcores
    bpw = b // nw
    ch = min(bpw, 64)
    nch = bpw // ch
    mesh = plsc.VectorSubcoreMesh(core_axis_name="c", subcore_axis_name="s")
    fn = pl.kernel(
        functools.partial(_sc_gather_body, nch, ch, bpw),
        mesh=mesh,
        out_type=jax.ShapeDtypeStruct((b, table.shape[1]), table.dtype),
        scratch_types=[
            pltpu.VMEM((ch,), jnp.int32),
            pltpu.VMEM((ch, table.shape[1]), table.dtype),
            pltpu.SemaphoreType.DMA,
        ],
    )
    return fn(table, idx)


def _ffn_body(be_ref, tb_ref, x_ref, w1_ref, w3_ref, w2_ref, y_ref):
    i = pl.program_id(0)

    @pl.when(i < tb_ref[0])
    def _():
        x = x_ref[...]
        h1 = jnp.dot(x, w1_ref[0], preferred_element_type=jnp.float32)
        h3 = jnp.dot(x, w3_ref[0], preferred_element_type=jnp.float32)
        h = h1 * (1.0 / (1.0 + jnp.exp(-h1))) * h3
        y_ref[...] = jnp.dot(h, w2_ref[0], preferred_element_type=jnp.float32)


def _ffn(x_pad, be, tb, W1, W3, W2):
    grid_spec = pltpu.PrefetchScalarGridSpec(
        num_scalar_prefetch=2,
        grid=(NB,),
        in_specs=[
            pl.BlockSpec((BT, D),
                         lambda i, be, tb: (jnp.where(i < tb[0], i, 0), 0)),
            pl.BlockSpec((1, D, F), lambda i, be, tb: (be[i], 0, 0)),
            pl.BlockSpec((1, D, F), lambda i, be, tb: (be[i], 0, 0)),
            pl.BlockSpec((1, F, D), lambda i, be, tb: (be[i], 0, 0)),
        ],
        out_specs=pl.BlockSpec((BT, D),
                               lambda i, be, tb: (jnp.where(i < tb[0], i, NB),
                                                  0)),
    )
    return pl.pallas_call(
        _ffn_body,
        grid_spec=grid_spec,
        out_shape=jax.ShapeDtypeStruct((NBT + BT, D), jnp.float32),
    )(be, tb, x_pad, W1, W3, W2)


def kernel(hidden_states, Wg, W1, W3, W2):
    b, s, d = hidden_states.shape
    x = hidden_states.reshape(-1, d)

    logits, pos2, bevec = _router(x, Wg)
    pos = pos2[:, 0]

    be = bevec[:NB, 0]
    tb = bevec[NB, 0][None]

    x_pad = _sc_scatter(x, pos, NBT)
    y_pad = _ffn(x_pad, be, tb, W1, W3, W2)
    out = _sc_gather(y_pad, pos.astype(jnp.int32))

    return out.reshape(b, s, d), logits
